# baseline (speedup 1.0000x reference)
import jax
import jax.numpy as jnp
from jax import lax
from jax.experimental import pallas as pl
from jax.experimental.pallas import tpu as pltpu, tpu_sc as plsc

N = 100000
C = 128
NC, NS, L = 2, 16, 16
NW = NC * NS
B0 = 80
NB = 4
NBLK = N // B0
NFULL = -(-NBLK // NW)
NLONG = NBLK - NW * (NFULL - 1)
G = B0 // L


def _body(inp_hbm, out_hbm, idx_all, *rest):
    bufs, sems, semi = rest[:NB], rest[NB:2 * NB], rest[2 * NB]
    c = lax.axis_index("c")
    s = lax.axis_index("s")
    wid = s * NC + c

    cnt = jnp.where(wid < NLONG, NFULL, NFULL - 1)
    start = (NFULL - 1) * wid + jnp.minimum(wid, NLONG)

    def idx_copy(nblocks):
        return pltpu.make_async_copy(
            inp_hbm.at[pl.ds(start * B0, nblocks * B0)],
            idx_all.at[pl.ds(0, nblocks * B0)], semi)

    @pl.when(wid < NLONG)
    def _():
        idx_copy(NFULL).start()

    @pl.when(wid >= NLONG)
    def _():
        idx_copy(NFULL - 1).start()

    iota = lax.iota(jnp.int32, 16)
    ones = jnp.ones((16,), jnp.int32)
    zeros = jnp.zeros((16,), jnp.int32)

    def scat(slot, buf, val):
        def one_group(g, cc):
            vals = idx_all[pl.ds(slot * B0 + g * L, L)]
            lin = (g * L + iota) * C + vals
            plsc.store_scatter(buf, [lin], val)
            return cc
        lax.fori_loop(0, G, one_group, 0)

    def zero_fill(buf):
        def one_chunk(j, cc):
            for u in range(8):
                buf[pl.ds(j * 128 + u * 16, 16)] = zeros
            return cc
        lax.fori_loop(0, B0 * C // 128, one_chunk, 0)

    def process(j, buf, sem):
        dst = out_hbm.at[pl.ds((start + j) * B0 * C, B0 * C)]

        @pl.when(j >= NB)
        def _():
            pltpu.make_async_copy(buf, dst, sem).wait()
            scat(j - NB, buf, zeros)

        @pl.when(j < NB)
        def _():
            zero_fill(buf)

        @pl.when(j == 0)
        def _():
            @pl.when(wid < NLONG)
            def _():
                idx_copy(NFULL).wait()

            @pl.when(wid >= NLONG)
            def _():
                idx_copy(NFULL - 1).wait()

        scat(j, buf, ones)
        pltpu.async_copy(buf, dst, sem)

    def do_block(j, carry):
        @pl.when(j < cnt)
        def _():
            for p in range(NB):
                @pl.when(j % NB == p)
                def _():
                    process(j, bufs[p], sems[p])

        return carry

    lax.fori_loop(0, NFULL, do_block, 0)

    anydst = out_hbm.at[pl.ds(0, B0 * C)]
    for p in range(NB):
        pltpu.make_async_copy(bufs[p], anydst, sems[p]).wait()


_onehot_sc = pl.kernel(
    _body,
    out_type=jax.ShapeDtypeStruct((N * C,), jnp.int32),
    mesh=plsc.VectorSubcoreMesh(core_axis_name="c", subcore_axis_name="s"),
    scratch_types=(
        [pltpu.VMEM((NFULL * B0,), jnp.int32)]
        + [pltpu.VMEM((B0 * C,), jnp.int32) for _ in range(NB)]
        + [pltpu.SemaphoreType.DMA for _ in range(NB + 1)]
    ),
    compiler_params=pltpu.CompilerParams(needs_layout_passes=False),
)


def kernel(inp):
    out = _onehot_sc(inp)
    return out.reshape(N, C)


# speedup vs baseline: 1.0342x; 1.0342x over previous
import jax
import jax.numpy as jnp
from jax import lax
from jax.experimental import pallas as pl
from jax.experimental.pallas import tpu as pltpu, tpu_sc as plsc

N = 100000
C = 128
NC, NS, L = 2, 16, 16
NW = NC * NS
B0 = 32
NB = 2
NBLK = N // B0
NFULL = -(-NBLK // NW)
NLONG = NBLK - NW * (NFULL - 1)
G = B0 // L


def _body(inp_hbm, out_hbm, idx_all, *rest):
    bufs, sems, semi = rest[:NB], rest[NB:2 * NB], rest[2 * NB]
    c = lax.axis_index("c")
    s = lax.axis_index("s")
    wid = s * NC + c

    cnt = jnp.where(wid < NLONG, NFULL, NFULL - 1)
    start = (NFULL - 1) * wid + jnp.minimum(wid, NLONG)

    def idx_copy(nblocks):
        return pltpu.make_async_copy(
            inp_hbm.at[pl.ds(start * B0, nblocks * B0)],
            idx_all.at[pl.ds(0, nblocks * B0)], semi)

    @pl.when(wid < NLONG)
    def _():
        idx_copy(NFULL).start()

    @pl.when(wid >= NLONG)
    def _():
        idx_copy(NFULL - 1).start()

    iota = lax.iota(jnp.int32, 16)
    ones = jnp.ones((16,), jnp.int32)
    zeros = jnp.zeros((16,), jnp.int32)

    def scat(slot, buf, val):
        def one_group(g, cc):
            vals = idx_all[pl.ds(slot * B0 + g * L, L)]
            lin = (g * L + iota) * C + vals
            plsc.store_scatter(buf, [lin], val)
            return cc
        lax.fori_loop(0, G, one_group, 0)

    def zero_fill(buf):
        def one_chunk(j, cc):
            for u in range(8):
                buf[pl.ds(j * 128 + u * 16, 16)] = zeros
            return cc
        lax.fori_loop(0, B0 * C // 128, one_chunk, 0)

    def process(j, buf, sem):
        dst = out_hbm.at[pl.ds((start + j) * B0 * C, B0 * C)]

        @pl.when(j >= NB)
        def _():
            pltpu.make_async_copy(buf, dst, sem).wait()
            scat(j - NB, buf, zeros)

        @pl.when(j < NB)
        def _():
            zero_fill(buf)

        @pl.when(j == 0)
        def _():
            @pl.when(wid < NLONG)
            def _():
                idx_copy(NFULL).wait()

            @pl.when(wid >= NLONG)
            def _():
                idx_copy(NFULL - 1).wait()

        scat(j, buf, ones)
        pltpu.async_copy(buf, dst, sem)

    def do_block(j, carry):
        @pl.when(j < cnt)
        def _():
            for p in range(NB):
                @pl.when(j % NB == p)
                def _():
                    process(j, bufs[p], sems[p])

        return carry

    lax.fori_loop(0, NFULL, do_block, 0)

    anydst = out_hbm.at[pl.ds(0, B0 * C)]
    for p in range(NB):
        pltpu.make_async_copy(bufs[p], anydst, sems[p]).wait()


_onehot_sc = pl.kernel(
    _body,
    out_type=jax.ShapeDtypeStruct((N * C,), jnp.int32),
    mesh=plsc.VectorSubcoreMesh(core_axis_name="c", subcore_axis_name="s"),
    scratch_types=(
        [pltpu.VMEM((NFULL * B0,), jnp.int32)]
        + [pltpu.VMEM((B0 * C,), jnp.int32) for _ in range(NB)]
        + [pltpu.SemaphoreType.DMA for _ in range(NB + 1)]
    ),
    compiler_params=pltpu.CompilerParams(needs_layout_passes=False),
)


def kernel(inp):
    out = _onehot_sc(inp)
    return out.reshape(N, C)
